# baseline (speedup 1.0000x reference)
import functools

import jax
import jax.numpy as jnp
from jax import lax
from jax.experimental import pallas as pl
from jax.experimental.pallas import tpu as pltpu
from jax.experimental.pallas import tpu_sc as plsc

B = 4096
D = 16
F = 26
UV = 1000000
SV = 100000
IN_FEAT = (F + 2) * D

NC = 2
NS = 16
NW = NC * NS
BPW = B // NW
L = 16


@functools.cache
def _sc_gather_fn():
    mesh = plsc.VectorSubcoreMesh(core_axis_name="c", subcore_axis_name="s")

    @functools.partial(
        pl.kernel,
        out_type=jax.ShapeDtypeStruct((IN_FEAT, B), jnp.float32),
        mesh=mesh,
        compiler_params=pltpu.CompilerParams(use_tc_tiling_on_sc=False),
        scratch_types=[
            pltpu.VMEM((BPW,), jnp.int32),
            pltpu.VMEM((BPW,), jnp.int32),
            pltpu.VMEM((F, BPW), jnp.int32),
            pltpu.VMEM((IN_FEAT, BPW), jnp.int32),
            pltpu.VMEM((IN_FEAT, BPW), jnp.float32),
            pltpu.SemaphoreType.DMA,
        ],
    )
    def _sc_gather(ut1, it1, sp1, uid, iid, sft, out_hbm,
                   uidv, iidv, sfv, idxb, outb, sem):
        wid = lax.axis_index("s") * NC + lax.axis_index("c")
        base = wid * BPW
        pltpu.sync_copy(uid.at[pl.ds(base, BPW)], uidv)
        pltpu.sync_copy(iid.at[pl.ds(base, BPW)], iidv)
        pltpu.sync_copy(sft.at[:, pl.ds(base, BPW)], sfv)

        def build_u(d, _):
            for k in range(BPW // L):
                idxb[d, pl.ds(L * k, L)] = uidv[pl.ds(L * k, L)] + d * UV
            return 0

        def build_i(d, _):
            for k in range(BPW // L):
                idxb[D + d, pl.ds(L * k, L)] = iidv[pl.ds(L * k, L)] + d * UV
            return 0

        def build_s(r, _):
            f = r // D
            for k in range(BPW // L):
                idxb[2 * D + r, pl.ds(L * k, L)] = sfv[f, pl.ds(L * k, L)] + r * SV
            return 0

        lax.fori_loop(0, D, build_u, 0)
        lax.fori_loop(0, D, build_i, 0)
        lax.fori_loop(0, F * D, build_s, 0)

        def fire_u(d, _):
            pltpu.async_copy(ut1.at[idxb.at[d]], outb.at[d], sem)
            return 0

        def fire_i(d, _):
            pltpu.async_copy(it1.at[idxb.at[D + d]], outb.at[D + d], sem)
            return 0

        def fire_s(r, _):
            pltpu.async_copy(sp1.at[idxb.at[2 * D + r]], outb.at[2 * D + r], sem)
            return 0

        lax.fori_loop(0, D, fire_u, 0)
        lax.fori_loop(0, D, fire_i, 0)
        lax.fori_loop(0, F * D, fire_s, 0)

        pltpu.make_async_copy(out_hbm.at[:, pl.ds(base, BPW)], outb, sem).wait()
        pltpu.sync_copy(outb, out_hbm.at[:, pl.ds(base, BPW)])

    return _sc_gather


def _dense_t_body(xt, K, cb, W0t, b0, W1t, b1, W2t, b2, Wot, bo, Wt, out):
    x0 = xt[...]
    dn = (((1,), (0,)), ((), ()))
    dot = lax.dot_general(K[0], x0, dn, preferred_element_type=jnp.float32) + cb[0]
    x1 = x0 * dot + x0
    dot = lax.dot_general(K[1], x1, dn, preferred_element_type=jnp.float32) + cb[1]
    x2 = x0 * dot + x1
    h = jnp.maximum(lax.dot_general(W0t[...], x0, dn, preferred_element_type=jnp.float32) + b0[...], 0.0)
    h = jnp.maximum(lax.dot_general(W1t[...], h, dn, preferred_element_type=jnp.float32) + b1[...], 0.0)
    h = jnp.maximum(lax.dot_general(W2t[...], h, dn, preferred_element_type=jnp.float32) + b2[...], 0.0)
    deep = lax.dot_general(Wot[...], h, dn, preferred_element_type=jnp.float32) + bo[...]
    stack = jnp.concatenate([x2, deep], axis=0)
    logit = lax.dot_general(stack, Wt[...], (((0,), (0,)), ((), ())),
                            preferred_element_type=jnp.float32)
    out[...] = 1.0 / (1.0 + jnp.exp(-logit))


def _dense_t_call(xt, K, cb, W0t, b0, W1t, b1, W2t, b2, Wot, bo, Wt):
    BB = 512
    grid = (B // BB,)
    full = lambda *s: pl.BlockSpec(s, lambda i: (0,) * len(s))
    return pl.pallas_call(
        _dense_t_body,
        grid=grid,
        in_specs=[
            pl.BlockSpec((IN_FEAT, BB), lambda i: (0, i)),
            full(2, IN_FEAT, IN_FEAT),
            full(2, IN_FEAT, 1),
            full(2 * D, IN_FEAT),
            full(2 * D, 1),
            full(2 * D, 2 * D),
            full(2 * D, 1),
            full(2 * D, 2 * D),
            full(2 * D, 1),
            full(D, 2 * D),
            full(D, 1),
            full(IN_FEAT + D, 1),
        ],
        out_specs=pl.BlockSpec((BB, 1), lambda i: (i, 0)),
        out_shape=jax.ShapeDtypeStruct((B, 1), jnp.float32),
    )(xt, K, cb, W0t, b0, W1t, b1, W2t, b2, Wot, bo, Wt)


def kernel(user_ids, item_ids, sparse_features, user_table, item_table,
           sparse_tables, kernels, cbias, W0, b0, W1, b1, W2, b2, Wo, bo, Wt):
    ut1 = user_table.T.reshape(-1)
    it1 = item_table.T.reshape(-1)
    sp1 = sparse_tables.transpose(0, 2, 1).reshape(-1)
    sft = sparse_features.T.astype(jnp.int32)
    comb_t = _sc_gather_fn()(ut1, it1, sp1,
                             user_ids.astype(jnp.int32),
                             item_ids.astype(jnp.int32), sft)
    return _dense_t_call(
        comb_t, kernels, cbias,
        W0.T, b0.reshape(2 * D, 1), W1.T, b1.reshape(2 * D, 1),
        W2.T, b2.reshape(2 * D, 1), Wo.T, bo.reshape(D, 1), Wt)


# speedup vs baseline: 7.2290x; 7.2290x over previous
import functools

import jax
import jax.numpy as jnp
from jax import lax
from jax.experimental import pallas as pl
from jax.experimental.pallas import tpu as pltpu
from jax.experimental.pallas import tpu_sc as plsc

B = 4096
D = 16
F = 26
UV = 1000000
SV = 100000
IN_FEAT = (F + 2) * D

NC = 2
NS = 16
NW = NC * NS
BPW = B // NW
L = 16

NT = (UV + 127) // 128
DT_CH = 1024
DT_FULL = UV // DT_CH
DT_TAIL0 = DT_FULL * DT_CH
DT_T1 = 512
DT_TAIL1 = DT_TAIL0 + DT_T1
DT_REM = UV - DT_TAIL1
NGRP = DT_FULL + 1


@functools.cache
def _sc_tilecopy_fn():
    mesh = plsc.VectorSubcoreMesh(core_axis_name="c", subcore_axis_name="s")

    @functools.partial(
        pl.kernel,
        out_type=(
            jax.ShapeDtypeStruct((2 * NT, 8, 128), jnp.float32),
            jax.ShapeDtypeStruct((2 * NT, 8, 128), jnp.float32),
        ),
        mesh=mesh,
        scratch_types=[
            pltpu.VMEM((8, DT_CH), jnp.float32),
            pltpu.VMEM((8 * DT_REM,), jnp.float32),
            pltpu.VMEM((8, 128), jnp.float32),
            pltpu.SemaphoreType.DMA,
            pltpu.SemaphoreType.DMA,
        ],
    )
    def _sc_tilecopy(ut2, it2, utail, itail, u_phys, i_phys,
                     buf, tb1, tb2, wsem, tsem):
        wid = lax.axis_index("s") * NC + lax.axis_index("c")

        def table(tab, tail, phys):
            def chunk(k, _):
                cid = wid + k * NW
                t = cid // NGRP
                jg = cid % NGRP

                @pl.when((cid < 2 * NGRP) & (jg < DT_FULL))
                def _():
                    pltpu.sync_copy(
                        tab.at[pl.ds(8 * t, 8), pl.ds(jg * DT_CH, DT_CH)], buf)
                    for j in range(DT_CH // 128):
                        pltpu.async_copy(
                            buf.at[:, pl.ds(128 * j, 128)],
                            phys.at[t * NT + 8 * jg + j], wsem)
                    pltpu.make_async_copy(
                        tab.at[pl.ds(0, 8), pl.ds(0, DT_CH)], buf, wsem).wait()

                @pl.when((cid < 2 * NGRP) & (jg == DT_FULL))
                def _():
                    pltpu.sync_copy(
                        tab.at[pl.ds(8 * t, 8), pl.ds(DT_TAIL0, DT_T1)],
                        buf.at[:, pl.ds(0, DT_T1)])
                    for j in range(DT_T1 // 128):
                        pltpu.async_copy(
                            buf.at[:, pl.ds(128 * j, 128)],
                            phys.at[t * NT + DT_FULL * 8 + j], tsem)
                    pltpu.sync_copy(tail.at[pl.ds(t * 8 * DT_REM, 8 * DT_REM)], tb1)
                    for r in range(8):
                        for k in range(DT_REM // L):
                            tb2[r, pl.ds(L * k, L)] = tb1[pl.ds(r * DT_REM + L * k, L)]
                    pltpu.async_copy(tb2, phys.at[t * NT + NT - 1], tsem)
                    pltpu.make_async_copy(
                        tab.at[pl.ds(0, 8), pl.ds(0, DT_T1)],
                        buf.at[:, pl.ds(0, DT_T1)], tsem).wait()
                    pltpu.make_async_copy(
                        tab.at[pl.ds(0, 8), pl.ds(0, 128)], tb2, tsem).wait()
                return 0

            lax.fori_loop(0, (2 * NGRP + NW - 1) // NW, chunk, 0)

        table(ut2, utail, u_phys)
        table(it2, itail, i_phys)

    return _sc_tilecopy


@functools.cache
def _sc_gather_fn():
    mesh = plsc.VectorSubcoreMesh(core_axis_name="c", subcore_axis_name="s")

    @functools.partial(
        pl.kernel,
        out_type=jax.ShapeDtypeStruct((IN_FEAT, B), jnp.float32),
        mesh=mesh,
        compiler_params=pltpu.CompilerParams(use_tc_tiling_on_sc=False),
        scratch_types=[
            pltpu.VMEM((BPW,), jnp.int32),
            pltpu.VMEM((BPW,), jnp.int32),
            pltpu.VMEM((BPW,), jnp.int32),
            pltpu.VMEM((BPW,), jnp.int32),
            pltpu.VMEM((F, BPW), jnp.int32),
            pltpu.VMEM((2 * D, BPW), jnp.int32),
            pltpu.VMEM((IN_FEAT, BPW), jnp.float32),
            pltpu.SemaphoreType.DMA,
        ],
    )
    def _sc_gather(u1, i1, sp1, uid, iid, sft, out_hbm,
                   uidv, iidv, uph, iph, sfv, idxb, outb, sem):
        wid = lax.axis_index("s") * NC + lax.axis_index("c")
        base = wid * BPW
        pltpu.sync_copy(uid.at[pl.ds(base, BPW)], uidv)
        pltpu.sync_copy(iid.at[pl.ds(base, BPW)], iidv)
        pltpu.sync_copy(sft.at[:, pl.ds(base, BPW)], sfv)

        def vph(ids_ref, out_ref, k, _=None):
            v = ids_ref[pl.ds(L * k, L)]
            out_ref[pl.ds(L * k, L)] = (
                lax.shift_left(lax.shift_right_logical(v, 7), 10)
                + jnp.bitwise_and(v, 127))
            return 0

        lax.fori_loop(0, BPW // L, functools.partial(vph, uidv, uph), 0)
        lax.fori_loop(0, BPW // L, functools.partial(vph, iidv, iph), 0)

        def build(d, _):
            c = (d // 8) * (NT * 1024) + (d % 8) * 128
            for k in range(BPW // L):
                idxb[d, pl.ds(L * k, L)] = uph[pl.ds(L * k, L)] + c
                idxb[D + d, pl.ds(L * k, L)] = iph[pl.ds(L * k, L)] + c
            return 0

        lax.fori_loop(0, D, build, 0)

        def fire_u(d, _):
            pltpu.async_copy(u1.at[idxb.at[d]], outb.at[d], sem)
            return 0

        def fire_i(d, _):
            pltpu.async_copy(i1.at[idxb.at[D + d]], outb.at[D + d], sem)
            return 0

        def fire_s(r, _):
            pltpu.async_copy(sp1.at[r].at[sfv.at[r // D]], outb.at[2 * D + r], sem)
            return 0

        lax.fori_loop(0, D, fire_u, 0)
        lax.fori_loop(0, D, fire_i, 0)
        lax.fori_loop(0, F * D, fire_s, 0)

        pltpu.make_async_copy(out_hbm.at[:, pl.ds(base, BPW)], outb, sem).wait()
        pltpu.sync_copy(outb, out_hbm.at[:, pl.ds(base, BPW)])

    return _sc_gather


def _dense_t_body(xt, K, cb, W0t, b0, W1t, b1, W2t, b2, Wot, bo, Wt, out):
    x0 = xt[...]
    dn = (((1,), (0,)), ((), ()))
    dot = lax.dot_general(K[0], x0, dn, preferred_element_type=jnp.float32) + cb[0]
    x1 = x0 * dot + x0
    dot = lax.dot_general(K[1], x1, dn, preferred_element_type=jnp.float32) + cb[1]
    x2 = x0 * dot + x1
    h = jnp.maximum(lax.dot_general(W0t[...], x0, dn, preferred_element_type=jnp.float32) + b0[...], 0.0)
    h = jnp.maximum(lax.dot_general(W1t[...], h, dn, preferred_element_type=jnp.float32) + b1[...], 0.0)
    h = jnp.maximum(lax.dot_general(W2t[...], h, dn, preferred_element_type=jnp.float32) + b2[...], 0.0)
    deep = lax.dot_general(Wot[...], h, dn, preferred_element_type=jnp.float32) + bo[...]
    stack = jnp.concatenate([x2, deep], axis=0)
    logit = lax.dot_general(stack, Wt[...], (((0,), (0,)), ((), ())),
                            preferred_element_type=jnp.float32)
    out[...] = 1.0 / (1.0 + jnp.exp(-logit))


def _dense_t_call(xt, K, cb, W0t, b0, W1t, b1, W2t, b2, Wot, bo, Wt):
    BB = 512
    grid = (B // BB,)
    full = lambda *s: pl.BlockSpec(s, lambda i: (0,) * len(s))
    return pl.pallas_call(
        _dense_t_body,
        grid=grid,
        in_specs=[
            pl.BlockSpec((IN_FEAT, BB), lambda i: (0, i)),
            full(2, IN_FEAT, IN_FEAT),
            full(2, IN_FEAT, 1),
            full(2 * D, IN_FEAT),
            full(2 * D, 1),
            full(2 * D, 2 * D),
            full(2 * D, 1),
            full(2 * D, 2 * D),
            full(2 * D, 1),
            full(D, 2 * D),
            full(D, 1),
            full(IN_FEAT + D, 1),
        ],
        out_specs=pl.BlockSpec((BB, 1), lambda i: (i, 0)),
        out_shape=jax.ShapeDtypeStruct((B, 1), jnp.float32),
    )(xt, K, cb, W0t, b0, W1t, b1, W2t, b2, Wot, bo, Wt)


def kernel(user_ids, item_ids, sparse_features, user_table, item_table,
           sparse_tables, kernels, cbias, W0, b0, W1, b1, W2, b2, Wo, bo, Wt):
    ut2 = user_table.T
    it2 = item_table.T
    utail = user_table.T[:, DT_TAIL1:].reshape(-1)
    itail = item_table.T[:, DT_TAIL1:].reshape(-1)
    u_phys, i_phys = _sc_tilecopy_fn()(ut2, it2, utail, itail)
    u1 = u_phys.reshape(-1)
    i1 = i_phys.reshape(-1)
    sp1 = sparse_tables.transpose(0, 2, 1).reshape(F * D, SV)
    sft = sparse_features.T.astype(jnp.int32)
    comb_t = _sc_gather_fn()(u1, i1, sp1,
                             user_ids.astype(jnp.int32),
                             item_ids.astype(jnp.int32), sft)
    return _dense_t_call(
        comb_t, kernels, cbias,
        W0.T, b0.reshape(2 * D, 1), W1.T, b1.reshape(2 * D, 1),
        W2.T, b2.reshape(2 * D, 1), Wo.T, bo.reshape(D, 1), Wt)
